# speedup vs baseline: 2.9760x; 2.9760x over previous
import functools

import jax
import jax.numpy as jnp
from jax import lax
from jax.experimental import pallas as pl
from jax.experimental.pallas import tpu as pltpu
from jax.experimental.pallas import tpu_sc as plsc

D = 128
C = 128


@jax.jit
def _embed(idx3, table):
    NW, n_chunks, _ = idx3.shape

    mesh = plsc.VectorSubcoreMesh(core_axis_name="c", subcore_axis_name="s")
    info = plsc.get_sparse_core_info()
    NC = info.num_cores

    @functools.partial(
        pl.kernel,
        out_type=jax.ShapeDtypeStruct((NW, n_chunks, C, D), jnp.float32),
        mesh=mesh,
        scratch_types=[
            pltpu.VMEM((n_chunks, C), jnp.int32),
            pltpu.VMEM((C, D), jnp.float32),
            pltpu.SemaphoreType.DMA,
        ],
    )
    def body(idx_hbm, table_hbm, out_hbm, idx_v, buf, sem):
        wid = lax.axis_index("s") * NC + lax.axis_index("c")
        pltpu.sync_copy(idx_hbm.at[wid], idx_v)

        @pl.loop(0, n_chunks)
        def _chunk(j):
            pltpu.async_copy(table_hbm.at[idx_v.at[j]], buf, sem).wait()
            pltpu.sync_copy(buf, out_hbm.at[wid, j])

    return body(idx3, table)


def kernel(input, table):
    B, H = input.shape
    N = B * H
    NW = 32
    n_per_w = N // NW
    n_chunks = n_per_w // C
    idx3 = input.reshape(NW, n_chunks, C).astype(jnp.int32)
    out = _embed(idx3, table)
    return out.reshape(B, H, D)


# speedup vs baseline: 3.3424x; 1.1231x over previous
import functools

import jax
import jax.numpy as jnp
from jax import lax
from jax.experimental import pallas as pl
from jax.experimental.pallas import tpu as pltpu
from jax.experimental.pallas import tpu_sc as plsc

D = 128
C = 128
NBUF = 5


@jax.jit
def _embed(idx3, table):
    NW, n_chunks, _ = idx3.shape

    mesh = plsc.VectorSubcoreMesh(core_axis_name="c", subcore_axis_name="s")
    info = plsc.get_sparse_core_info()
    NC = info.num_cores

    @functools.partial(
        pl.kernel,
        out_type=jax.ShapeDtypeStruct((NW, n_chunks, C, D), jnp.float32),
        mesh=mesh,
        scratch_types=(
            [pltpu.VMEM((n_chunks, C), jnp.int32)]
            + [pltpu.VMEM((C, D), jnp.float32) for _ in range(NBUF)]
            + [pltpu.SemaphoreType.DMA for _ in range(2 * NBUF)]
        ),
    )
    def body(idx_hbm, table_hbm, out_hbm, idx_v, *rest):
        bufs = rest[:NBUF]
        gsem = rest[NBUF:2 * NBUF]
        osem = rest[2 * NBUF:]
        wid = lax.axis_index("s") * NC + lax.axis_index("c")
        pltpu.sync_copy(idx_hbm.at[wid], idx_v)

        for b in range(NBUF):
            pltpu.async_copy(table_hbm.at[idx_v.at[b]], bufs[b], gsem[b])

        @pl.loop(0, n_chunks - NBUF, step=NBUF)
        def _steady(j0):
            for b in range(NBUF):
                j = j0 + b
                pltpu.make_async_copy(
                    table_hbm.at[idx_v.at[b]], bufs[b], gsem[b]).wait()
                pltpu.async_copy(bufs[b], out_hbm.at[wid, j], osem[b])
                pltpu.make_async_copy(
                    bufs[b], out_hbm.at[wid, j], osem[b]).wait()
                pltpu.async_copy(
                    table_hbm.at[idx_v.at[j + NBUF]], bufs[b], gsem[b])

        for b in range(NBUF):
            j = n_chunks - NBUF + b
            pltpu.make_async_copy(
                table_hbm.at[idx_v.at[b]], bufs[b], gsem[b]).wait()
            pltpu.async_copy(bufs[b], out_hbm.at[wid, j], osem[b])
            pltpu.make_async_copy(
                bufs[b], out_hbm.at[wid, j], osem[b]).wait()

    return body(idx3, table)


def kernel(input, table):
    B, H = input.shape
    N = B * H
    NW = 32
    n_per_w = N // NW
    n_chunks = n_per_w // C
    idx3 = input.reshape(NW, n_chunks, C).astype(jnp.int32)
    out = _embed(idx3, table)
    return out.reshape(B, H, D)
